# baseline (speedup 1.0000x reference)
import functools

import jax
import jax.numpy as jnp
from jax import lax
from jax.experimental import pallas as pl
from jax.experimental.pallas import tpu as pltpu
from jax.experimental.pallas import tpu_sc as plsc

N_NODES = 10000
N_EDGES = 320000
D = 128
NUM_SC = 2
NUM_TILES = 16
NUM_WORKERS = NUM_SC * NUM_TILES
E_PER_TILE = N_EDGES // NUM_WORKERS
CHUNK = 80
NCHUNK = 126
E_PAD = NCHUNK * CHUNK
SUPER = 3
NSUPER = NCHUNK // SUPER
SE = SUPER * CHUNK
N_PAD = 10240
ROWS_PER_TILE = N_PAD // NUM_TILES

_mesh = plsc.VectorSubcoreMesh(
    core_axis_name="c", subcore_axis_name="s",
    num_cores=NUM_SC, num_subcores=NUM_TILES,
)


@functools.partial(
    pl.kernel,
    out_type=jax.ShapeDtypeStruct((NUM_SC, N_PAD, D), jnp.float32),
    mesh=_mesh,
    scratch_types=[
        [pltpu.VMEM((1, CHUNK), jnp.int32) for _ in range(6)],
        [pltpu.VMEM((SE,), jnp.int32) for _ in range(2)],
        [pltpu.VMEM((SE,), jnp.float32) for _ in range(2)],
        [pltpu.VMEM((CHUNK, D), jnp.float32) for _ in range(3)],
        pltpu.VMEM_SHARED((N_PAD, D), jnp.float32),
        [pltpu.SemaphoreType.DMA for _ in range(6)],
        [pltpu.SemaphoreType.DMA for _ in range(2)],
        [pltpu.SemaphoreType.DMA for _ in range(3)],
        [pltpu.SemaphoreType.DMA for _ in range(3)],
    ],
)
def _spmm_sc(x_hbm, row_hbm, col_hbm, val_hbm, zero_hbm, out_hbm,
             rbufs, cbufs, vbufs, fbufs, acc, rsems, cvsems, gsems, ssems):
    c = lax.axis_index("c")
    s = lax.axis_index("s")
    wid = c * NUM_TILES + s
    base = wid * E_PAD

    pltpu.sync_copy(zero_hbm, acc.at[pl.ds(s * ROWS_PER_TILE, ROWS_PER_TILE)])
    plsc.subcore_barrier()

    def issue_row(j, p):
        pltpu.async_copy(row_hbm.at[wid, j], rbufs[p], rsems[p])

    def wait_row(p):
        pltpu.make_async_copy(row_hbm.at[wid, 0], rbufs[p], rsems[p]).wait()

    def issue_cv(sup, rot):
        sl = pl.ds(base + sup * SE, SE)
        pltpu.async_copy(col_hbm.at[sl], cbufs[rot], cvsems[rot])
        pltpu.async_copy(val_hbm.at[sl], vbufs[rot], cvsems[rot])

    def wait_cv(rot):
        sl = pl.ds(base, SE)
        pltpu.make_async_copy(col_hbm.at[sl], cbufs[rot], cvsems[rot]).wait()
        pltpu.make_async_copy(val_hbm.at[sl], vbufs[rot], cvsems[rot]).wait()

    def issue_gather(k3, rot, qs):
        pltpu.async_copy(
            x_hbm.at[cbufs[rot].at[pl.ds(qs * CHUNK, CHUNK)]],
            fbufs[k3], gsems[k3])

    def wait_gather(k3, rot, qs):
        pltpu.make_async_copy(
            x_hbm.at[cbufs[rot].at[pl.ds(qs * CHUNK, CHUNK)]],
            fbufs[k3], gsems[k3]).wait()

    def issue_scatter(k3, p):
        pltpu.async_copy(
            fbufs[k3], acc.at[rbufs[p].at[0]], ssems[k3], add=True)

    def wait_scatter(k3, p):
        pltpu.make_async_copy(
            fbufs[k3], acc.at[rbufs[p].at[0]], ssems[k3]).wait()

    def scale(k3, rot, qs):
        buf, vals = fbufs[k3], vbufs[rot]

        @pl.loop(0, CHUNK // 16)
        def _grp(g):
            vv = vals[pl.ds(qs * CHUNK + g * 16, 16)]

            @pl.loop(0, 16, unroll=4)
            def _row(r2):
                vb = vv.at[jnp.full((16,), r2, jnp.int32)].get(
                    mode="promise_in_bounds")
                r = g * 16 + r2
                for q in range(D // 16):
                    buf[r, pl.ds(q * 16, 16)] = buf[r, pl.ds(q * 16, 16)] * vb

    def chunk_body(j, ph, *, ws=True, ie=True, icv=True, wcv=True, ig=True):
        k3 = ph % 3
        p = ph % 6
        rot = (ph // 3) % 2
        if ws:
            wait_scatter((ph + 1) % 3, (ph + 4) % 6)
        if ie:
            issue_row(j + 3, (ph + 3) % 6)
        if icv and k3 == 0:
            issue_cv(j // 3 + 1, (rot + 1) % 2)
        if wcv and k3 == 2:
            wait_cv((rot + 1) % 2)
        if ig:
            wait_row((ph + 1) % 6)
            issue_gather((ph + 1) % 3, ((ph + 1) // 3) % 2, (ph + 1) % 3)
        wait_gather(k3, rot, k3)
        scale(k3, rot, k3)
        issue_scatter(k3, p)

    issue_row(0, 0)
    issue_row(1, 1)
    issue_row(2, 2)
    issue_cv(0, 0)
    issue_cv(1, 1)
    wait_cv(0)
    wait_row(0)
    issue_gather(0, 0, 0)

    chunk_body(0, 6, ws=False, icv=False)
    chunk_body(1, 7, ws=False)
    for t in range(2, 6):
        chunk_body(t, 6 + t)

    @pl.loop(6, NCHUNK - 6, step=6)
    def _six(J):
        for t in range(6):
            chunk_body(J + t, 6 + t)

    for t in range(120, 126):
        chunk_body(t, 6 + t % 6,
                   ie=(t < 123), icv=(t != 123), wcv=(t != 125),
                   ig=(t != 125))
    wait_scatter((NCHUNK - 2) % 3, (NCHUNK - 2) % 6)
    wait_scatter((NCHUNK - 1) % 3, (NCHUNK - 1) % 6)

    plsc.subcore_barrier()
    pltpu.sync_copy(acc.at[pl.ds(s * ROWS_PER_TILE, ROWS_PER_TILE)],
                    out_hbm.at[c, pl.ds(s * ROWS_PER_TILE, ROWS_PER_TILE)])


def _combine_tc(p_ref, w_ref, b_ref, o_ref):
    agg = p_ref[0] + p_ref[1]
    o_ref[...] = (
        jnp.dot(agg, w_ref[...], preferred_element_type=jnp.float32)
        + b_ref[...]
    )


_BLK_M = 2000


def kernel(x, adj_mat_indices, adj_mat_values, weights, bias):
    pad = ((0, 0), (0, E_PAD - E_PER_TILE))
    row = jnp.pad(adj_mat_indices[0].reshape(NUM_WORKERS, E_PER_TILE), pad)
    col = jnp.pad(adj_mat_indices[1].reshape(NUM_WORKERS, E_PER_TILE), pad)
    val = jnp.pad(adj_mat_values.reshape(NUM_WORKERS, E_PER_TILE), pad)
    row4 = row.reshape(NUM_WORKERS, NCHUNK, 1, CHUNK)
    zero = jnp.zeros((ROWS_PER_TILE, D), jnp.float32)
    parts = _spmm_sc(x, row4, col.reshape(-1), val.reshape(-1), zero)
    return pl.pallas_call(
        _combine_tc,
        grid=(N_NODES // _BLK_M,),
        in_specs=[
            pl.BlockSpec((NUM_SC, _BLK_M, D), lambda i: (0, i, 0)),
            pl.BlockSpec((D, D), lambda i: (0, 0)),
            pl.BlockSpec((1, D), lambda i: (0, 0)),
        ],
        out_specs=pl.BlockSpec((_BLK_M, D), lambda i: (i, 0)),
        out_shape=jax.ShapeDtypeStruct((N_NODES, D), jnp.float32),
    )(parts, weights, bias.reshape(1, D))


# speedup vs baseline: 1.6397x; 1.6397x over previous
import functools

import jax
import jax.numpy as jnp
from jax import lax
from jax.experimental import pallas as pl
from jax.experimental.pallas import tpu as pltpu
from jax.experimental.pallas import tpu_sc as plsc

N_NODES = 10000
N_EDGES = 320000
D = 128
NUM_SC = 2
NUM_TILES = 16
NUM_WORKERS = NUM_SC * NUM_TILES
E_PER_TILE = N_EDGES // NUM_WORKERS
CHUNK = 80
NCHUNK = E_PER_TILE // CHUNK
N_PAD = 10240
ROWS_PER_TILE = N_PAD // NUM_TILES

_mesh = plsc.VectorSubcoreMesh(
    core_axis_name="c", subcore_axis_name="s",
    num_cores=NUM_SC, num_subcores=NUM_TILES,
)


@functools.partial(
    pl.kernel,
    out_type=jax.ShapeDtypeStruct((NUM_SC, N_PAD, D), jnp.float32),
    mesh=_mesh,
    scratch_types=[
        [pltpu.VMEM((1, CHUNK), jnp.int32) for _ in range(6)],
        [pltpu.VMEM((CHUNK,), jnp.int32) for _ in range(6)],
        [pltpu.VMEM((CHUNK,), jnp.float32) for _ in range(6)],
        [pltpu.VMEM((CHUNK, D), jnp.float32) for _ in range(3)],
        pltpu.VMEM_SHARED((N_PAD, D), jnp.float32),
        [pltpu.SemaphoreType.DMA for _ in range(6)],
        [pltpu.SemaphoreType.DMA for _ in range(3)],
        [pltpu.SemaphoreType.DMA for _ in range(3)],
    ],
)
def _spmm_sc(x_hbm, row_hbm, col_hbm, val_hbm, out_hbm,
             rbufs, cbufs, vbufs, fbufs, acc, esems, gsems, ssems):
    c = lax.axis_index("c")
    s = lax.axis_index("s")
    wid = c * NUM_TILES + s
    base = wid * E_PER_TILE

    @pl.loop(0, CHUNK)
    def _zrow(r):
        for q in range(D // 16):
            fbufs[0][r, pl.ds(q * 16, 16)] = jnp.zeros((16,), jnp.float32)
    for i in range(ROWS_PER_TILE // CHUNK):
        pltpu.sync_copy(
            fbufs[0], acc.at[pl.ds(s * ROWS_PER_TILE + i * CHUNK, CHUNK)])
    plsc.subcore_barrier()

    def issue_edges(j, p):
        sl = pl.ds(base + j * CHUNK, CHUNK)
        pltpu.async_copy(row_hbm.at[wid, j], rbufs[p], esems[p])
        pltpu.async_copy(col_hbm.at[sl], cbufs[p], esems[p])
        pltpu.async_copy(val_hbm.at[sl], vbufs[p], esems[p])

    def wait_edges(p):
        sl = pl.ds(base, CHUNK)
        pltpu.make_async_copy(row_hbm.at[wid, 0], rbufs[p], esems[p]).wait()
        pltpu.make_async_copy(col_hbm.at[sl], cbufs[p], esems[p]).wait()
        pltpu.make_async_copy(val_hbm.at[sl], vbufs[p], esems[p]).wait()

    def issue_gather(k3, p):
        pltpu.async_copy(x_hbm.at[cbufs[p]], fbufs[k3], gsems[k3])

    def wait_gather(k3, p):
        pltpu.make_async_copy(x_hbm.at[cbufs[p]], fbufs[k3], gsems[k3]).wait()

    def issue_scatter(k3, p):
        pltpu.async_copy(
            fbufs[k3], acc.at[rbufs[p].at[0]], ssems[k3], add=True)

    def wait_scatter(k3, p):
        pltpu.make_async_copy(
            fbufs[k3], acc.at[rbufs[p].at[0]], ssems[k3]).wait()

    def scale(k3, p):
        buf, vals = fbufs[k3], vbufs[p]

        @pl.loop(0, CHUNK // 16)
        def _grp(g):
            vv = vals[pl.ds(g * 16, 16)]

            @pl.loop(0, 16, unroll=4)
            def _row(r2):
                vb = vv.at[jnp.full((16,), r2, jnp.int32)].get(
                    mode="promise_in_bounds")
                r = g * 16 + r2
                for q in range(D // 16):
                    buf[r, pl.ds(q * 16, 16)] = buf[r, pl.ds(q * 16, 16)] * vb

    def chunk_body(j, jph, *, ws=True, ie=True, ig=True):
        k3, p = jph % 3, jph % 6
        if ws:
            wait_scatter((jph + 1) % 3, (jph + 4) % 6)
        if ie:
            if isinstance(j, int):
                if j + 3 < NCHUNK:
                    issue_edges(j + 3, (jph + 3) % 6)
            else:
                @pl.when(j + 3 < NCHUNK)
                def _ie():
                    issue_edges(j + 3, (jph + 3) % 6)
        if ig:
            wait_edges((jph + 1) % 6)
            issue_gather((jph + 1) % 3, (jph + 1) % 6)
        wait_gather(k3, p)
        scale(k3, p)
        issue_scatter(k3, p)

    issue_edges(0, 0)
    issue_edges(1, 1)
    issue_edges(2, 2)
    wait_edges(0)
    issue_gather(0, 0)
    chunk_body(0, 0, ws=False)
    chunk_body(1, 1, ws=False)
    chunk_body(2, 2)

    @pl.loop(3, NCHUNK - 2, step=6)
    def _six(J):
        for t in range(6):
            chunk_body(J + t, 3 + t)

    chunk_body(NCHUNK - 2, NCHUNK - 2, ie=False)
    chunk_body(NCHUNK - 1, NCHUNK - 1, ie=False, ig=False)
    wait_scatter((NCHUNK - 2) % 3, (NCHUNK - 2) % 6)
    wait_scatter((NCHUNK - 1) % 3, (NCHUNK - 1) % 6)

    plsc.subcore_barrier()
    pltpu.sync_copy(acc.at[pl.ds(s * ROWS_PER_TILE, ROWS_PER_TILE)],
                    out_hbm.at[c, pl.ds(s * ROWS_PER_TILE, ROWS_PER_TILE)])


def _combine_tc(p_ref, w_ref, b_ref, o_ref):
    agg = p_ref[0] + p_ref[1]
    o_ref[...] = (
        jnp.dot(agg, w_ref[...], preferred_element_type=jnp.float32)
        + b_ref[...]
    )


_BLK_M = 2000


def kernel(x, adj_mat_indices, adj_mat_values, weights, bias):
    row4 = adj_mat_indices[0].reshape(NUM_WORKERS, NCHUNK, 1, CHUNK)
    parts = _spmm_sc(x, row4, adj_mat_indices[1], adj_mat_values)
    return pl.pallas_call(
        _combine_tc,
        grid=(N_NODES // _BLK_M,),
        in_specs=[
            pl.BlockSpec((NUM_SC, _BLK_M, D), lambda i: (0, i, 0)),
            pl.BlockSpec((D, D), lambda i: (0, 0)),
            pl.BlockSpec((1, D), lambda i: (0, 0)),
        ],
        out_specs=pl.BlockSpec((_BLK_M, D), lambda i: (i, 0)),
        out_shape=jax.ShapeDtypeStruct((N_NODES, D), jnp.float32),
    )(parts, weights, bias.reshape(1, D))


# speedup vs baseline: 1.6720x; 1.0197x over previous
import functools

import jax
import jax.numpy as jnp
from jax import lax
from jax.experimental import pallas as pl
from jax.experimental.pallas import tpu as pltpu
from jax.experimental.pallas import tpu_sc as plsc

N_NODES = 10000
N_EDGES = 320000
D = 128
NUM_SC = 2
NUM_TILES = 16
NUM_WORKERS = NUM_SC * NUM_TILES
E_PER_TILE = N_EDGES // NUM_WORKERS
CHUNK = 80
NCHUNK = E_PER_TILE // CHUNK
N_PAD = 10240
ROWS_PER_TILE = N_PAD // NUM_TILES

_mesh = plsc.VectorSubcoreMesh(
    core_axis_name="c", subcore_axis_name="s",
    num_cores=NUM_SC, num_subcores=NUM_TILES,
)


@functools.partial(
    pl.kernel,
    out_type=jax.ShapeDtypeStruct((NUM_SC, N_PAD, D), jnp.float32),
    mesh=_mesh,
    scratch_types=[
        [pltpu.VMEM((1, CHUNK), jnp.int32) for _ in range(6)],
        [pltpu.VMEM((CHUNK,), jnp.int32) for _ in range(6)],
        [pltpu.VMEM((CHUNK,), jnp.float32) for _ in range(6)],
        [pltpu.VMEM((CHUNK, D), jnp.float32) for _ in range(4)],
        pltpu.VMEM_SHARED((N_PAD, D), jnp.float32),
        [pltpu.SemaphoreType.DMA for _ in range(6)],
        [pltpu.SemaphoreType.DMA for _ in range(4)],
        [pltpu.SemaphoreType.DMA for _ in range(4)],
    ],
)
def _spmm_sc(x_hbm, row_hbm, col_hbm, val_hbm, out_hbm,
             rbufs, cbufs, vbufs, fbufs, acc, esems, gsems, ssems):
    c = lax.axis_index("c")
    s = lax.axis_index("s")
    wid = c * NUM_TILES + s
    base = wid * E_PER_TILE

    @pl.loop(0, CHUNK)
    def _zrow(r):
        for q in range(D // 16):
            fbufs[0][r, pl.ds(q * 16, 16)] = jnp.zeros((16,), jnp.float32)
    for i in range(ROWS_PER_TILE // CHUNK):
        pltpu.sync_copy(
            fbufs[0], acc.at[pl.ds(s * ROWS_PER_TILE + i * CHUNK, CHUNK)])
    plsc.subcore_barrier()

    def issue_edges(j, p):
        sl = pl.ds(base + j * CHUNK, CHUNK)
        pltpu.async_copy(row_hbm.at[wid, j], rbufs[p], esems[p])
        pltpu.async_copy(col_hbm.at[sl], cbufs[p], esems[p])
        pltpu.async_copy(val_hbm.at[sl], vbufs[p], esems[p])

    def wait_edges(p):
        sl = pl.ds(base, CHUNK)
        pltpu.make_async_copy(row_hbm.at[wid, 0], rbufs[p], esems[p]).wait()
        pltpu.make_async_copy(col_hbm.at[sl], cbufs[p], esems[p]).wait()
        pltpu.make_async_copy(val_hbm.at[sl], vbufs[p], esems[p]).wait()

    def issue_gather(k4, p):
        pltpu.async_copy(x_hbm.at[cbufs[p]], fbufs[k4], gsems[k4])

    def wait_gather(k4, p):
        pltpu.make_async_copy(x_hbm.at[cbufs[p]], fbufs[k4], gsems[k4]).wait()

    def issue_scatter(k4, p):
        pltpu.async_copy(
            fbufs[k4], acc.at[rbufs[p].at[0]], ssems[k4], add=True)

    def wait_scatter(k4, p):
        pltpu.make_async_copy(
            fbufs[k4], acc.at[rbufs[p].at[0]], ssems[k4]).wait()

    def scale(k4, p):
        buf, vals = fbufs[k4], vbufs[p]

        @pl.loop(0, CHUNK // 16)
        def _grp(g):
            vv = vals[pl.ds(g * 16, 16)]

            @pl.loop(0, 16, unroll=4)
            def _row(r2):
                vb = vv.at[jnp.full((16,), r2, jnp.int32)].get(
                    mode="promise_in_bounds")
                r = g * 16 + r2
                for q in range(D // 16):
                    buf[r, pl.ds(q * 16, 16)] = buf[r, pl.ds(q * 16, 16)] * vb

    def chunk_body(j, ph, *, ws=True, ie=True, ig=True):
        k4, p = ph % 4, ph % 6
        if ws:
            wait_scatter((ph + 2) % 4, (ph + 4) % 6)
        if ie:
            issue_edges(j + 4, (ph + 4) % 6)
        if ig:
            wait_edges((ph + 2) % 6)
            issue_gather((ph + 2) % 4, (ph + 2) % 6)
        wait_gather(k4, p)
        scale(k4, p)
        issue_scatter(k4, p)

    issue_edges(0, 0)
    issue_edges(1, 1)
    issue_edges(2, 2)
    issue_edges(3, 3)
    wait_edges(0)
    issue_gather(0, 0)
    wait_edges(1)
    issue_gather(1, 1)
    chunk_body(0, 12, ws=False)
    chunk_body(1, 13, ws=False)
    for t in range(2, 8):
        chunk_body(t, 12 + t)

    @pl.loop(8, NCHUNK - 9, step=12)
    def _twelve(J):
        for t in range(12):
            chunk_body(J + t, 12 + (8 + t) % 12)

    for t in range(116, 125):
        chunk_body(t, 12 + t % 12, ie=(t + 4 < NCHUNK), ig=(t + 2 < NCHUNK))
    wait_scatter((NCHUNK - 2) % 4, (NCHUNK - 2) % 6)
    wait_scatter((NCHUNK - 1) % 4, (NCHUNK - 1) % 6)

    plsc.subcore_barrier()
    pltpu.sync_copy(acc.at[pl.ds(s * ROWS_PER_TILE, ROWS_PER_TILE)],
                    out_hbm.at[c, pl.ds(s * ROWS_PER_TILE, ROWS_PER_TILE)])


def _combine_tc(p_ref, w_ref, b_ref, o_ref):
    agg = p_ref[0] + p_ref[1]
    o_ref[...] = (
        jnp.dot(agg, w_ref[...], preferred_element_type=jnp.float32)
        + b_ref[...]
    )


_BLK_M = 2000


def kernel(x, adj_mat_indices, adj_mat_values, weights, bias):
    row4 = adj_mat_indices[0].reshape(NUM_WORKERS, NCHUNK, 1, CHUNK)
    parts = _spmm_sc(x, row4, adj_mat_indices[1], adj_mat_values)
    return pl.pallas_call(
        _combine_tc,
        grid=(N_NODES // _BLK_M,),
        in_specs=[
            pl.BlockSpec((NUM_SC, _BLK_M, D), lambda i: (0, i, 0)),
            pl.BlockSpec((D, D), lambda i: (0, 0)),
            pl.BlockSpec((1, D), lambda i: (0, 0)),
        ],
        out_specs=pl.BlockSpec((_BLK_M, D), lambda i: (i, 0)),
        out_shape=jax.ShapeDtypeStruct((N_NODES, D), jnp.float32),
    )(parts, weights, bias.reshape(1, D))
